# baseline (speedup 1.0000x reference)
import functools

import jax
import jax.numpy as jnp
from jax import lax
from jax.experimental import pallas as pl
from jax.experimental.pallas import tpu as pltpu
from jax.experimental.pallas import tpu_sc as plsc

N = 100000
D = 128
NFEAT = 17

NC = 2
NS = 16
NW = NC * NS

C = 200
NCHUNK = N // C
CHUNKS_PER_W = (NCHUNK + NW - 1) // NW


def _proj_body(x_ref, w_ref, b_ref, o_ref):
    xf = x_ref[...].astype(jnp.float32)
    o_ref[...] = (
        jnp.dot(xf, w_ref[...], preferred_element_type=jnp.float32) + b_ref[...]
    )


_BN = 800

_proj_call = pl.pallas_call(
    _proj_body,
    grid=(N // _BN,),
    in_specs=[
        pl.BlockSpec((_BN, NFEAT), lambda i: (i, 0)),
        pl.BlockSpec((NFEAT, D), lambda i: (0, 0)),
        pl.BlockSpec((1, D), lambda i: (0, 0)),
    ],
    out_specs=pl.BlockSpec((_BN, D), lambda i: (i, 0)),
    out_shape=jax.ShapeDtypeStruct((N, D), jnp.float32),
)


_mesh = plsc.VectorSubcoreMesh(core_axis_name="c", subcore_axis_name="s")


@functools.partial(
    pl.kernel,
    mesh=_mesh,
    out_type=jax.ShapeDtypeStruct((N, D), jnp.float32),
    scratch_types=[
        pltpu.VMEM((C,), jnp.int32),
        pltpu.VMEM((C, D), jnp.float32),
        pltpu.VMEM((C, D), jnp.float32),
        pltpu.SemaphoreType.DMA,
    ],
)
def _gather_add(idx_hbm, table_hbm, proj_hbm, out_hbm, idx_v, emb_v, acc_v, sem):
    wid = lax.axis_index("s") * NC + lax.axis_index("c")

    def chunk_body(t, carry):
        cid = wid + t * NW

        @pl.when(cid < NCHUNK)
        def _():
            base = cid * C
            pltpu.sync_copy(idx_hbm.at[pl.ds(base, C)], idx_v)
            gat = pltpu.async_copy(table_hbm.at[idx_v], emb_v, sem)
            pltpu.sync_copy(proj_hbm.at[pl.ds(base, C)], acc_v)
            gat.wait()

            def row_body(r, c2):
                for j in range(D // 16):
                    sl = pl.ds(j * 16, 16)
                    acc_v[r, sl] = acc_v[r, sl] + emb_v[r, sl]
                return c2

            lax.fori_loop(0, C, row_body, 0)
            pltpu.sync_copy(acc_v, out_hbm.at[pl.ds(base, C)])

        return carry

    lax.fori_loop(0, CHUNKS_PER_W, chunk_body, 0)


def kernel(x, type_embed, W, b):
    idx = x[:, 0]
    Wp = jnp.concatenate([jnp.zeros((1, D), W.dtype), W], axis=0)
    proj = _proj_call(x, Wp, b.reshape(1, D))
    return _gather_add(idx, type_embed, proj)


# speedup vs baseline: 1.2237x; 1.2237x over previous
import functools

import jax
import jax.numpy as jnp
from jax import lax
from jax.experimental import pallas as pl
from jax.experimental.pallas import tpu as pltpu
from jax.experimental.pallas import tpu_sc as plsc

N = 100000
D = 128
NFEAT = 17

HALF = N // 2

NC = 2
NS = 16
NW = NC * NS

C = 200
NCHUNK = HALF // C
TMAX = (NCHUNK + NW - 1) // NW


def _make_sc_gather(base_row):
    mesh = plsc.VectorSubcoreMesh(core_axis_name="c", subcore_axis_name="s")

    @functools.partial(
        pl.kernel,
        mesh=mesh,
        out_type=jax.ShapeDtypeStruct((HALF, D), jnp.float32),
        scratch_types=[
            pltpu.VMEM((C,), jnp.int32),
            pltpu.VMEM((C,), jnp.int32),
            pltpu.VMEM((C, D), jnp.float32),
            pltpu.VMEM((C, D), jnp.float32),
            pltpu.SemaphoreType.DMA,
            pltpu.SemaphoreType.DMA,
            pltpu.SemaphoreType.DMA,
            pltpu.SemaphoreType.DMA,
        ],
    )
    def sc_gather(idx_hbm, table_hbm, out_hbm,
                  idx0, idx1, emb0, emb1,
                  gsem0, gsem1, wsem0, wsem1):
        wid = lax.axis_index("s") * NC + lax.axis_index("c")
        idxs = (idx0, idx1)
        embs = (emb0, emb1)
        gsems = (gsem0, gsem1)
        wsems = (wsem0, wsem1)

        def start(t):
            s = t % 2
            cid = wid + t * NW

            @pl.when(cid < NCHUNK)
            def _():
                row0 = base_row + cid * C
                pltpu.sync_copy(idx_hbm.at[pl.ds(row0, C)], idxs[s])
                pltpu.async_copy(table_hbm.at[idxs[s]], embs[s], gsems[s])

        def finish(t):
            s = t % 2
            cid = wid + t * NW

            @pl.when(cid < NCHUNK)
            def _():
                pltpu.make_async_copy(table_hbm.at[idxs[s]], embs[s], gsems[s]).wait()
                pltpu.async_copy(embs[s], out_hbm.at[pl.ds(cid * C, C)], wsems[s])

        def drain(t):
            s = t % 2
            cid = wid + t * NW

            @pl.when(cid < NCHUNK)
            def _():
                pltpu.make_async_copy(
                    embs[s], out_hbm.at[pl.ds(cid * C, C)], wsems[s]
                ).wait()

        start(0)
        for t in range(TMAX):
            if t + 1 < TMAX:
                if t - 1 >= 0:
                    drain(t - 1)
                start(t + 1)
            finish(t)
        for t in range(max(TMAX - 2, 0), TMAX):
            drain(t)

    return sc_gather


_BI = 2048


def _idx_body(x_ref, o_ref):
    o_ref[...] = x_ref[...][:, 0]


_idx_call = pl.pallas_call(
    _idx_body,
    grid=((N + _BI - 1) // _BI,),
    in_specs=[pl.BlockSpec((_BI, NFEAT), lambda i: (i, 0))],
    out_specs=pl.BlockSpec((_BI,), lambda i: (i,)),
    out_shape=jax.ShapeDtypeStruct((N,), jnp.int32),
)


_BN = 2000
_NB = HALF // _BN


def _tc_body_a(x_ref, g_ref, w_ref, b_ref, o_ref):
    xf = x_ref[...].astype(jnp.float32)
    o_ref[...] = (
        jnp.dot(xf, w_ref[...], preferred_element_type=jnp.float32)
        + b_ref[...]
        + g_ref[...]
    )


def _tc_body_b(part_ref, x_ref, g_ref, w_ref, b_ref, o_ref):
    del part_ref
    xf = x_ref[...].astype(jnp.float32)
    o_ref[...] = (
        jnp.dot(xf, w_ref[...], preferred_element_type=jnp.float32)
        + b_ref[...]
        + g_ref[...]
    )


_tc_a = pl.pallas_call(
    _tc_body_a,
    grid=(_NB,),
    in_specs=[
        pl.BlockSpec((_BN, NFEAT), lambda i: (i, 0)),
        pl.BlockSpec((_BN, D), lambda i: (i, 0)),
        pl.BlockSpec((NFEAT, D), lambda i: (0, 0)),
        pl.BlockSpec((1, D), lambda i: (0, 0)),
    ],
    out_specs=pl.BlockSpec((_BN, D), lambda i: (i, 0)),
    out_shape=jax.ShapeDtypeStruct((N, D), jnp.float32),
)

_tc_b = pl.pallas_call(
    _tc_body_b,
    grid=(_NB,),
    in_specs=[
        pl.BlockSpec(memory_space=pltpu.HBM),
        pl.BlockSpec((_BN, NFEAT), lambda i: (i + _NB, 0)),
        pl.BlockSpec((_BN, D), lambda i: (i, 0)),
        pl.BlockSpec((NFEAT, D), lambda i: (0, 0)),
        pl.BlockSpec((1, D), lambda i: (0, 0)),
    ],
    out_specs=pl.BlockSpec((_BN, D), lambda i: (i + _NB, 0)),
    out_shape=jax.ShapeDtypeStruct((N, D), jnp.float32),
    input_output_aliases={0: 0},
)

_sc_gather_lo = _make_sc_gather(0)
_sc_gather_hi = _make_sc_gather(HALF)


def kernel(x, type_embed, W, b):
    Wp = jnp.concatenate([jnp.zeros((1, D), W.dtype), W], axis=0)
    b2 = b.reshape(1, D)
    idx = _idx_call(x)
    g_lo = _sc_gather_lo(idx, type_embed)
    g_hi = _sc_gather_hi(idx, type_embed)
    part = _tc_a(x, g_lo, Wp, b2)
    return _tc_b(part, x, g_hi, Wp, b2)


# speedup vs baseline: 2.1337x; 1.7436x over previous
import functools

import jax
import jax.numpy as jnp
from jax import lax
from jax.experimental import pallas as pl
from jax.experimental.pallas import tpu as pltpu
from jax.experimental.pallas import tpu_sc as plsc

N = 100000
D = 128
NFEAT = 17

_BN = 2048
NBA = 25
A_ROWS = NBA * _BN
B_ROWS = N - A_ROWS
NBB = (B_ROWS + _BN - 1) // _BN

NC = 2
NS = 16
NW = NC * NS

C = 200


def _make_sc_gather(base_row, nrows):
    nchunk = nrows // C
    tmax = (nchunk + NW - 1) // NW
    mesh = plsc.VectorSubcoreMesh(core_axis_name="c", subcore_axis_name="s")

    @functools.partial(
        pl.kernel,
        mesh=mesh,
        out_type=jax.ShapeDtypeStruct((nrows, D), jnp.float32),
        scratch_types=[
            pltpu.VMEM((C,), jnp.int32),
            pltpu.VMEM((C,), jnp.int32),
            pltpu.VMEM((C, D), jnp.float32),
            pltpu.VMEM((C, D), jnp.float32),
            pltpu.SemaphoreType.DMA,
            pltpu.SemaphoreType.DMA,
            pltpu.SemaphoreType.DMA,
            pltpu.SemaphoreType.DMA,
        ],
    )
    def sc_gather(idx_hbm, table_hbm, out_hbm,
                  idx0, idx1, emb0, emb1,
                  gsem0, gsem1, wsem0, wsem1):
        wid = lax.axis_index("s") * NC + lax.axis_index("c")
        idxs = (idx0, idx1)
        embs = (emb0, emb1)
        gsems = (gsem0, gsem1)
        wsems = (wsem0, wsem1)

        def start(t):
            s = t % 2
            cid = wid + t * NW

            @pl.when(cid < nchunk)
            def _():
                pltpu.sync_copy(
                    idx_hbm.at[pl.ds(base_row + cid * C, C)], idxs[s])
                pltpu.async_copy(table_hbm.at[idxs[s]], embs[s], gsems[s])

        def finish(t):
            s = t % 2
            cid = wid + t * NW

            @pl.when(cid < nchunk)
            def _():
                pltpu.make_async_copy(
                    table_hbm.at[idxs[s]], embs[s], gsems[s]).wait()
                pltpu.async_copy(
                    embs[s], out_hbm.at[pl.ds(cid * C, C)], wsems[s])

        def drain(t):
            s = t % 2
            cid = wid + t * NW

            @pl.when(cid < nchunk)
            def _():
                pltpu.make_async_copy(
                    embs[s], out_hbm.at[pl.ds(cid * C, C)], wsems[s]).wait()

        start(0)
        for t in range(tmax):
            if t + 1 < tmax:
                if t - 1 >= 0:
                    drain(t - 1)
                start(t + 1)
            finish(t)
        for t in range(max(tmax - 2, 0), tmax):
            drain(t)

    return sc_gather


def _tc_body_a(xt_ref, g_ref, w_ref, b_ref, o_ref):
    xf = xt_ref[...].astype(jnp.float32)
    mm = lax.dot_general(
        xf, w_ref[...], (((0,), (0,)), ((), ())),
        preferred_element_type=jnp.float32,
    )
    o_ref[...] = mm + b_ref[...] + g_ref[...]


def _tc_body_b(part_ref, xt_ref, g_ref, w_ref, b_ref, o_ref):
    del part_ref
    _tc_body_a(xt_ref, g_ref, w_ref, b_ref, o_ref)


_tc_a = pl.pallas_call(
    _tc_body_a,
    grid=(NBA,),
    in_specs=[
        pl.BlockSpec((NFEAT, _BN), lambda i: (0, i)),
        pl.BlockSpec((_BN, D), lambda i: (i, 0)),
        pl.BlockSpec((NFEAT, D), lambda i: (0, 0)),
        pl.BlockSpec((1, D), lambda i: (0, 0)),
    ],
    out_specs=pl.BlockSpec((_BN, D), lambda i: (i, 0)),
    out_shape=jax.ShapeDtypeStruct((N, D), jnp.float32),
)

_tc_b = pl.pallas_call(
    _tc_body_b,
    grid=(NBB,),
    in_specs=[
        pl.BlockSpec(memory_space=pltpu.HBM),
        pl.BlockSpec((NFEAT, _BN), lambda i: (0, i + NBA)),
        pl.BlockSpec((_BN, D), lambda i: (i, 0)),
        pl.BlockSpec((NFEAT, D), lambda i: (0, 0)),
        pl.BlockSpec((1, D), lambda i: (0, 0)),
    ],
    out_specs=pl.BlockSpec((_BN, D), lambda i: (i + NBA, 0)),
    out_shape=jax.ShapeDtypeStruct((N, D), jnp.float32),
    input_output_aliases={0: 0},
)

_sc_gather_lo = _make_sc_gather(0, A_ROWS)
_sc_gather_hi = _make_sc_gather(A_ROWS, B_ROWS)


def kernel(x, type_embed, W, b):
    xt = x.T
    idx = xt[0]
    Wp = jnp.concatenate([jnp.zeros((1, D), W.dtype), W], axis=0)
    b2 = b.reshape(1, D)
    g_lo = _sc_gather_lo(idx, type_embed)
    g_hi = _sc_gather_hi(idx, type_embed)
    part = _tc_a(xt, g_lo, Wp, b2)
    return _tc_b(part, xt, g_hi, Wp, b2)
